# baseline (speedup 1.0000x reference)
import functools

import jax
import jax.numpy as jnp
from jax import lax
from jax.experimental import pallas as pl
from jax.experimental.pallas import tpu as pltpu
from jax.experimental.pallas import tpu_sc as plsc

VOCAB_SIZE = 50304
HIDDEN = 2048
W = HIDDEN
NUM_CORES = 2
NUM_SUBCORES = 16
NUM_WORKERS = NUM_CORES * NUM_SUBCORES
BATCH = 4
SEQ = 4096
N_TOKENS = BATCH * SEQ
BPW = N_TOKENS // NUM_WORKERS
PAIRS = BPW // 2
PBLK = 8
TBLK = 2 * PBLK
NBLK = PAIRS // PBLK
W_PER_ROW = SEQ // BPW
UNROLL = 16
LANES = 16
MASK16 = 0xFFFF


def _make_kernel():
  mesh = plsc.VectorSubcoreMesh(
      core_axis_name="c", subcore_axis_name="s",
      num_cores=NUM_CORES, num_subcores=NUM_SUBCORES)

  @functools.partial(
      pl.kernel,
      mesh=mesh,
      out_type=jax.ShapeDtypeStruct((BATCH, SEQ, HIDDEN), jnp.float16),
      scratch_types=[
          pltpu.VMEM((BPW,), jnp.int32),
          pltpu.VMEM((BPW,), jnp.int32),
          pltpu.VMEM((BPW,), jnp.int32),
          pltpu.VMEM((TBLK, W), jnp.int32),
          pltpu.VMEM((TBLK, W), jnp.int32),
          pltpu.VMEM((PBLK, W), jnp.int32),
          pltpu.VMEM((PBLK, W), jnp.int32),
          pltpu.SemaphoreType.DMA,
          pltpu.SemaphoreType.DMA,
          pltpu.SemaphoreType.DMA,
          pltpu.SemaphoreType.DMA,
      ],
  )
  def body(idx_hbm, table_hbm, out_hbm, idx_v, pv, sh, g0, g1, ob0, ob1,
           sem0, sem1, ssem0, ssem1):
    wid = lax.axis_index("s") * NUM_CORES + lax.axis_index("c")
    b = wid // W_PER_ROW
    col = pl.multiple_of((wid % W_PER_ROW) * BPW, BPW)
    qcol = pl.multiple_of((wid % W_PER_ROW) * PAIRS, PAIRS)

    t32 = table_hbm.bitcast(jnp.int32)
    o32 = out_hbm.bitcast(jnp.int32)

    pltpu.sync_copy(idx_hbm.at[b, pl.ds(col, BPW)], idx_v)
    for i in range(BPW // LANES):
      v = idx_v[pl.ds(LANES * i, LANES)]
      pv[pl.ds(LANES * i, LANES)] = lax.shift_right_logical(v, 1)
      sh[pl.ds(LANES * i, LANES)] = lax.shift_left(v & 1, 4)

    def start(blk, buf, sem):
      off = pl.multiple_of(blk * TBLK, TBLK)
      return pltpu.async_copy(t32.at[pv.at[pl.ds(off, TBLK)]], buf, sem)

    def out_rows(blk):
      qrow = pl.multiple_of(qcol + blk * PBLK, PBLK)
      return o32.at[b, pl.ds(qrow, PBLK)]

    def merge_store(blk, gbuf, obuf, ssem):
      shvec = sh[pl.ds(pl.multiple_of(blk * TBLK, TBLK), TBLK)]
      for p in range(PBLK):
        s0v = shvec[2 * p]
        s1v = shvec[2 * p + 1]

        def cbody(ci, _, p=p, s0v=s0v, s1v=s1v):
          for u in range(UNROLL):
            off = pl.multiple_of((ci * UNROLL + u) * LANES, LANES)
            a = gbuf[2 * p, pl.ds(off, LANES)]
            bb = gbuf[2 * p + 1, pl.ds(off, LANES)]
            lo = lax.shift_right_logical(a, s0v) & MASK16
            hi = lax.shift_left(lax.shift_right_logical(bb, s1v), 16)
            obuf[p, pl.ds(off, LANES)] = lo | hi
          return 0

        lax.fori_loop(0, W // (LANES * UNROLL), cbody, 0)
      pltpu.async_copy(obuf, out_rows(blk), ssem)

    cp0 = start(0, g0, sem0)
    cp1 = start(1, g1, sem1)

    def block_pair(k2, _):
      blk_a = 2 * k2
      blk_b = 2 * k2 + 1
      cp0.wait()

      @pl.when(blk_a >= 2)
      def _():
        pltpu.make_async_copy(ob0, out_rows(blk_a - 2), ssem0).wait()

      merge_store(blk_a, g0, ob0, ssem0)

      @pl.when(blk_a + 2 < NBLK)
      def _():
        start(blk_a + 2, g0, sem0)

      cp1.wait()

      @pl.when(blk_b >= 3)
      def _():
        pltpu.make_async_copy(ob1, out_rows(blk_b - 2), ssem1).wait()

      merge_store(blk_b, g1, ob1, ssem1)

      @pl.when(blk_b + 2 < NBLK)
      def _():
        start(blk_b + 2, g1, sem1)

      return 0

    lax.fori_loop(0, NBLK // 2, block_pair, 0)
    pltpu.make_async_copy(ob0, out_rows(NBLK - 2), ssem0).wait()
    pltpu.make_async_copy(ob1, out_rows(NBLK - 1), ssem1).wait()

  return body


_embed = _make_kernel()


@jax.jit
def kernel(input_, weight):
  return _embed(input_.astype(jnp.int32), weight)


# speedup vs baseline: 1.0510x; 1.0510x over previous
import functools

import jax
import jax.numpy as jnp
from jax import lax
from jax.experimental import pallas as pl
from jax.experimental.pallas import tpu as pltpu
from jax.experimental.pallas import tpu_sc as plsc

VOCAB_SIZE = 50304
HIDDEN = 2048
W = HIDDEN
NUM_CORES = 2
NUM_SUBCORES = 16
NUM_WORKERS = NUM_CORES * NUM_SUBCORES
BATCH = 4
SEQ = 4096
N_TOKENS = BATCH * SEQ
BPW = N_TOKENS // NUM_WORKERS
PAIRS = BPW // 2
PBLK = 8
TBLK = 2 * PBLK
NBLK = PAIRS // PBLK
W_PER_ROW = SEQ // BPW
UNROLL = 8
LANES = 16
MASK16 = 0xFFFF


def _make_kernel():
  mesh = plsc.VectorSubcoreMesh(
      core_axis_name="c", subcore_axis_name="s",
      num_cores=NUM_CORES, num_subcores=NUM_SUBCORES)

  @functools.partial(
      pl.kernel,
      mesh=mesh,
      out_type=jax.ShapeDtypeStruct((BATCH, SEQ, HIDDEN), jnp.float16),
      scratch_types=[
          pltpu.VMEM((BPW,), jnp.int32),
          pltpu.VMEM((BPW,), jnp.int32),
          pltpu.VMEM((BPW,), jnp.int32),
          pltpu.VMEM((TBLK, W), jnp.int32),
          pltpu.VMEM((TBLK, W), jnp.int32),
          pltpu.VMEM((PBLK, W), jnp.int32),
          pltpu.VMEM((PBLK, W), jnp.int32),
          pltpu.SemaphoreType.DMA,
          pltpu.SemaphoreType.DMA,
          pltpu.SemaphoreType.DMA,
          pltpu.SemaphoreType.DMA,
      ],
  )
  def body(idx_hbm, table_hbm, out_hbm, idx_v, pv, sh, g0, g1, ob0, ob1,
           sem0, sem1, ssem0, ssem1):
    wid = lax.axis_index("s") * NUM_CORES + lax.axis_index("c")
    b = wid // W_PER_ROW
    col = pl.multiple_of((wid % W_PER_ROW) * BPW, BPW)
    qcol = pl.multiple_of((wid % W_PER_ROW) * PAIRS, PAIRS)

    t32 = table_hbm.bitcast(jnp.int32)
    o32 = out_hbm.bitcast(jnp.int32)

    pltpu.sync_copy(idx_hbm.at[b, pl.ds(col, BPW)], idx_v)
    for i in range(BPW // LANES):
      v = idx_v[pl.ds(LANES * i, LANES)]
      pv[pl.ds(LANES * i, LANES)] = lax.shift_right_logical(v, 1)
      sh[pl.ds(LANES * i, LANES)] = lax.shift_left(v & 1, 4)

    def start(blk, buf, sem):
      off = pl.multiple_of(blk * TBLK, TBLK)
      return pltpu.async_copy(t32.at[pv.at[pl.ds(off, TBLK)]], buf, sem)

    def out_rows(blk):
      qrow = pl.multiple_of(qcol + blk * PBLK, PBLK)
      return o32.at[b, pl.ds(qrow, PBLK)]

    def merge_store(blk, gbuf, obuf, ssem):
      shvec = sh[pl.ds(pl.multiple_of(blk * TBLK, TBLK), TBLK)]
      for p in range(PBLK):
        s0v = shvec[2 * p]
        s1v = shvec[2 * p + 1]

        def cbody(ci, _, p=p, s0v=s0v, s1v=s1v):
          for u in range(UNROLL):
            off = pl.multiple_of((ci * UNROLL + u) * LANES, LANES)
            a = gbuf[2 * p, pl.ds(off, LANES)]
            bb = gbuf[2 * p + 1, pl.ds(off, LANES)]
            lo = lax.shift_right_logical(a, s0v) & MASK16
            hi = lax.shift_left(lax.shift_right_logical(bb, s1v), 16)
            obuf[p, pl.ds(off, LANES)] = lo | hi
          return 0

        lax.fori_loop(0, W // (LANES * UNROLL), cbody, 0)
      pltpu.async_copy(obuf, out_rows(blk), ssem)

    cp0 = start(0, g0, sem0)
    cp1 = start(1, g1, sem1)

    def block_pair(k2, _):
      blk_a = 2 * k2
      blk_b = 2 * k2 + 1
      cp0.wait()

      @pl.when(blk_a >= 2)
      def _():
        pltpu.make_async_copy(ob0, out_rows(blk_a - 2), ssem0).wait()

      merge_store(blk_a, g0, ob0, ssem0)

      @pl.when(blk_a + 2 < NBLK)
      def _():
        start(blk_a + 2, g0, sem0)

      cp1.wait()

      @pl.when(blk_b >= 3)
      def _():
        pltpu.make_async_copy(ob1, out_rows(blk_b - 2), ssem1).wait()

      merge_store(blk_b, g1, ob1, ssem1)

      @pl.when(blk_b + 2 < NBLK)
      def _():
        start(blk_b + 2, g1, sem1)

      return 0

    lax.fori_loop(0, NBLK // 2, block_pair, 0)
    pltpu.make_async_copy(ob0, out_rows(NBLK - 2), ssem0).wait()
    pltpu.make_async_copy(ob1, out_rows(NBLK - 1), ssem1).wait()

  return body


_embed = _make_kernel()


@jax.jit
def kernel(input_, weight):
  return _embed(input_.astype(jnp.int32), weight)
